# baseline (speedup 1.0000x reference)
import jax
import jax.numpy as jnp
from jax.experimental import pallas as pl
from jax.experimental.pallas import tpu as pltpu


def _half(cb, tb, wf, bfc, wfc1, bfc1):
    h = (jnp.dot(cb, wf[:64, :], preferred_element_type=jnp.float32)
         + jnp.dot(tb, wf[64:, :], preferred_element_type=jnp.float32)
         + bfc)
    h = jnp.where(h > 0, h, jnp.exp(jnp.minimum(h, 0.0)) - 1.0)
    h = jnp.dot(h, wfc1, preferred_element_type=jnp.float32) + bfc1
    w = jax.nn.sigmoid(h)
    return tb + (cb - tb) * w, w


def _gate_body(ca_ref, cb_ref, ta_ref, tb_ref,
               wfc_ref, bfc_ref, wfc1_ref, bfc1_ref,
               m_ref, w_ref):
    wf = wfc_ref[...]
    bfc = bfc_ref[...]
    wfc1 = wfc1_ref[...]
    bfc1 = bfc1_ref[...]
    BN = ca_ref.shape[0]
    m_a, w_a = _half(ca_ref[...], ta_ref[...], wf, bfc, wfc1, bfc1)
    m_ref[:BN], w_ref[:BN] = m_a, w_a
    m_b, w_b = _half(cb_ref[...], tb_ref[...], wf, bfc, wfc1, bfc1)
    m_ref[BN:], w_ref[BN:] = m_b, w_b


def kernel(c, t, W_fc, b_fc, W_fc1, b_fc1):
    bs, n, dim = c.shape
    bfc2 = b_fc.reshape(1, dim)
    bfc12 = b_fc1.reshape(1, dim)

    BN = 5000
    grid = (bs, n // (2 * BN))

    spec_a = pl.BlockSpec((None, BN, dim), lambda b, i: (b, 2 * i, 0))
    spec_b = pl.BlockSpec((None, BN, dim), lambda b, i: (b, 2 * i + 1, 0))
    out_spec = pl.BlockSpec((None, 2 * BN, dim), lambda b, i: (b, i, 0))
    rep = lambda shape: pl.BlockSpec(shape, lambda b, i: (0, 0))

    mixed, w = pl.pallas_call(
        _gate_body,
        grid=grid,
        in_specs=[
            spec_a, spec_b, spec_a, spec_b,
            rep((2 * dim, dim)),
            rep((1, dim)),
            rep((dim, dim)),
            rep((1, dim)),
        ],
        out_specs=[out_spec, out_spec],
        out_shape=[
            jax.ShapeDtypeStruct((bs, n, dim), jnp.float32),
            jax.ShapeDtypeStruct((bs, n, dim), jnp.float32),
        ],
        compiler_params=pltpu.CompilerParams(
            dimension_semantics=("parallel", "parallel"),
        ),
    )(c, c, t, t, W_fc, bfc2, W_fc1, bfc12)

    return mixed, w


# speedup vs baseline: 1.0006x; 1.0006x over previous
import jax
import jax.numpy as jnp
from jax.experimental import pallas as pl
from jax.experimental.pallas import tpu as pltpu


def _gate_body(wd_ref, bd_ref, wd1_ref, bd1_ref,
               c_hbm, t_hbm, m_hbm, w_hbm):
    rows = c_hbm.size // 64
    cv = c_hbm.reshape(rows, 64)
    tv = t_hbm.reshape(rows, 64)
    mv = m_hbm.reshape(rows, 64)
    wv = w_hbm.reshape(rows, 64)

    wd = wd_ref[...]
    bd = bd_ref[...]
    wd1 = wd1_ref[...]
    bd1 = bd1_ref[...]

    BR = 8000
    grid = (rows // BR,)
    spec = pl.BlockSpec((BR, 64), lambda i: (i, 0))

    def inner(c_ref, t_ref, m_ref, w_ref):
        cb = c_ref[...]
        tb = t_ref[...]
        h = (jnp.dot(cb, wd[:64], preferred_element_type=jnp.float32)
             + jnp.dot(tb, wd[64:], preferred_element_type=jnp.float32)
             + bd)
        h = jnp.where(h > 0, h, jnp.exp(jnp.minimum(h, 0.0)) - 1.0)
        h = jnp.dot(h, wd1, preferred_element_type=jnp.float32) + bd1
        w = jax.nn.sigmoid(h)
        w_ref[...] = w
        m_ref[...] = tb + (cb - tb) * w

    pltpu.emit_pipeline(
        inner,
        grid=grid,
        in_specs=[spec, spec],
        out_specs=[spec, spec],
    )(cv, tv, mv, wv)


def kernel(c, t, W_fc, b_fc, W_fc1, b_fc1):
    bs, n, dim = c.shape

    Wd = W_fc
    Wd1 = W_fc1
    bd = b_fc.reshape(1, dim)
    bd1 = b_fc1.reshape(1, dim)

    rep = lambda shape: pl.BlockSpec(shape, lambda: (0, 0))
    any_spec = pl.BlockSpec(memory_space=pl.ANY)

    mixed, w = pl.pallas_call(
        _gate_body,
        in_specs=[
            rep((2 * dim, dim)),
            rep((1, dim)),
            rep((dim, dim)),
            rep((1, dim)),
            any_spec, any_spec,
        ],
        out_specs=[any_spec, any_spec],
        out_shape=[
            jax.ShapeDtypeStruct((bs, n, dim), jnp.float32),
            jax.ShapeDtypeStruct((bs, n, dim), jnp.float32),
        ],
    )(Wd, bd, Wd1, bd1, c, t)

    return mixed, w
